# baseline (speedup 1.0000x reference)
import functools

import jax
import jax.numpy as jnp
from jax import lax
from jax.experimental import pallas as pl
from jax.experimental.pallas import tpu as pltpu
from jax.experimental.pallas import tpu_sc as plsc

EMBED_DIM = 32
GROUP = 4
NUM_CORES = 2
NUM_SUBCORES = 16
NUM_WORKERS = NUM_CORES * NUM_SUBCORES
NBUF = 2
LANES = 16
CHUNK = 256


@jax.jit
def _gather(idx_t, table_g):
    S, H, B = idx_t.shape
    n_pairs = S * H
    pairs_per_w = n_pairs // NUM_WORKERS
    n_chunks = pairs_per_w * (B // CHUNK)
    mesh = plsc.VectorSubcoreMesh(core_axis_name="c", subcore_axis_name="s")

    @functools.partial(
        pl.kernel,
        mesh=mesh,
        out_type=jax.ShapeDtypeStruct((B, S, H, EMBED_DIM), jnp.float32),
        scratch_types=[
            pltpu.VMEM((NBUF, CHUNK), jnp.int32),
            pltpu.VMEM((NBUF, CHUNK), jnp.int32),
        ]
        + [pltpu.VMEM((CHUNK, GROUP * EMBED_DIM), jnp.float32)] * NBUF
        + [pltpu.VMEM((CHUNK, EMBED_DIM), jnp.float32)] * NBUF
        + [pltpu.SemaphoreType.DMA] * (3 * NBUF),
        compiler_params=pltpu.CompilerParams(use_tc_tiling_on_sc=False),
    )
    def k(idx_hbm, table_hbm, out_hbm, idx_v, idxhi_v, *bufs_and_sems):
        grp_v = bufs_and_sems[0:NBUF]
        row_v = bufs_and_sems[NBUF:2 * NBUF]
        sems = bufs_and_sems[2 * NBUF:]
        isem = sems[0:NBUF]
        gsem = sems[NBUF:2 * NBUF]
        osem = sems[2 * NBUF:3 * NBUF]
        wid = lax.axis_index("s") * NUM_CORES + lax.axis_index("c")
        pair0 = wid * pairs_per_w
        per_pair = B // CHUNK

        def coords(c):
            p = pair0 + c // per_pair
            b0 = (c % per_pair) * CHUNK
            return p // H, p % H, b0

        def idx_copy(c, b):
            s, h, b0 = coords(c)
            return pltpu.make_async_copy(
                idx_hbm.at[s, h, pl.ds(b0, CHUNK)], idx_v.at[b], isem[b])

        def gat_copy(b):
            return pltpu.make_async_copy(
                table_hbm.at[idxhi_v.at[b]], grp_v[b], gsem[b])

        def out_copy(c, b):
            s, h, b0 = coords(c)
            return pltpu.make_async_copy(
                row_v[b], out_hbm.at[pl.ds(b0, CHUNK), s, h], osem[b])

        def compute_hi(b):
            for i in range(CHUNK // LANES):
                sl = pl.ds(i * LANES, LANES)
                idxhi_v[b, sl] = lax.shift_right_logical(
                    idx_v[b, sl], GROUP // 2)

        def select_rows(b):
            grp_b = grp_v[b]
            row_b = row_v[b]

            def body(g, carry):
                j0 = g * LANES
                subs = (idx_v[b, pl.ds(j0, LANES)] & (GROUP - 1)) * EMBED_DIM
                for j in range(LANES):
                    sub = subs[j]
                    row = j0 + j
                    row_b[row, pl.ds(0, LANES)] = grp_b[row, pl.ds(sub, LANES)]
                    row_b[row, pl.ds(LANES, LANES)] = (
                        grp_b[row, pl.ds(sub + LANES, LANES)])
                return carry

            lax.fori_loop(0, CHUNK // LANES, body, 0)

        for b in range(NBUF):
            idx_copy(b, b).start()

        def chunk_body(c2, carry):
            for b in range(NBUF):
                c = c2 * NBUF + b

                @pl.when(c2 > 0)
                def _():
                    out_copy(c - NBUF, b).wait()

                idx_copy(c, b).wait()
                compute_hi(b)
                gat_copy(b).start()
                gat_copy(b).wait()
                select_rows(b)

                @pl.when(c + NBUF < n_chunks)
                def _():
                    idx_copy(c + NBUF, b).start()

                out_copy(c, b).start()
            return carry

        lax.fori_loop(0, n_chunks // NBUF, chunk_body, 0)
        for b in range(NBUF):
            out_copy(n_chunks - NBUF + b, b).wait()

    return k(idx_t, table_g)


def kernel(indices, table):
    idx_t = jnp.transpose(indices, (1, 2, 0))
    table_g = table.reshape(table.shape[0] // GROUP, GROUP * EMBED_DIM)
    return _gather(idx_t, table_g)


# speedup vs baseline: 1.2984x; 1.2984x over previous
import functools

import jax
import jax.numpy as jnp
from jax import lax
from jax.experimental import pallas as pl
from jax.experimental.pallas import tpu as pltpu
from jax.experimental.pallas import tpu_sc as plsc

EMBED_DIM = 32
TOTAL_ROWS = 1000000
NUM_CORES = 2
NUM_SUBCORES = 16
NUM_WORKERS = NUM_CORES * NUM_SUBCORES
NBUF = 2
SPLIT_B = 4


@jax.jit
def _gather(idx_t, table):
    S, H, B = idx_t.shape
    n_pairs = S * H
    pairs_per_w = n_pairs // NUM_WORKERS
    chunk = B // SPLIT_B
    n_chunks = pairs_per_w * SPLIT_B
    mesh = plsc.VectorSubcoreMesh(core_axis_name="c", subcore_axis_name="s")

    @functools.partial(
        pl.kernel,
        mesh=mesh,
        out_type=jax.ShapeDtypeStruct((B, S, H, EMBED_DIM), jnp.float32),
        scratch_types=[
            pltpu.VMEM((NBUF, chunk), jnp.int32),
            pltpu.VMEM((NBUF, chunk, EMBED_DIM), jnp.float32),
        ]
        + [pltpu.SemaphoreType.DMA] * (3 * NBUF),
        compiler_params=pltpu.CompilerParams(use_tc_tiling_on_sc=False),
    )
    def k(idx_hbm, table_hbm, out_hbm, idx_v, rows_v, *sems):
        isem = sems[0:NBUF]
        gsem = sems[NBUF:2 * NBUF]
        osem = sems[2 * NBUF:3 * NBUF]
        wid = lax.axis_index("s") * NUM_CORES + lax.axis_index("c")
        pair0 = wid * pairs_per_w

        def chunk_coords(c):
            p = pair0 + c // SPLIT_B
            b0 = (c % SPLIT_B) * chunk
            return p // H, p % H, b0

        idx_h, g_h, o_h = {}, {}, {}

        def idx_start(c):
            b = c % NBUF
            s, h, b0 = chunk_coords(c)
            idx_h[c] = pltpu.async_copy(
                idx_hbm.at[s, h, pl.ds(b0, chunk)], idx_v.at[b], isem[b])

        def gather_start(c):
            b = c % NBUF
            g_h[c] = pltpu.async_copy(
                table_hbm.at[idx_v.at[b]], rows_v.at[b], gsem[b])

        def out_start(c):
            b = c % NBUF
            s, h, b0 = chunk_coords(c)
            o_h[c] = pltpu.async_copy(
                rows_v.at[b], out_hbm.at[pl.ds(b0, chunk), s, h], osem[b])

        for c in range(min(NBUF, n_chunks)):
            idx_start(c)
        for c in range(n_chunks):
            if c >= NBUF:
                o_h[c - NBUF].wait()
            idx_h[c].wait()
            gather_start(c)
            g_h[c].wait()
            if c + NBUF < n_chunks:
                idx_start(c + NBUF)
            out_start(c)
        for c in range(max(0, n_chunks - NBUF), n_chunks):
            o_h[c].wait()

    return k(idx_t, table)


def kernel(indices, table):
    idx_t = jnp.transpose(indices, (1, 2, 0))
    return _gather(idx_t, table)
